# baseline (speedup 1.0000x reference)
import functools
import math

import jax
import jax.numpy as jnp
from jax import lax
from jax.experimental import pallas as pl
from jax.experimental.pallas import tpu as pltpu
from jax.experimental.pallas import tpu_sc as plsc

_N = 2048
_SIZE = 32000
_SMOOTHING = 0.1
_CONF = 1.0 - _SMOOTHING
_FILL = _SMOOTHING / (_SIZE - 2)
_ROW_K = _CONF * math.log(_CONF) + (_SIZE - 2) * _FILL * math.log(_FILL)

_NC = 2
_NS = 16
_NW = _NC * _NS
_R = _N // _NW
_L = 16
_TPR = _SIZE // 128


def _sc_body(xs_hbm, t_hbm, out_hbm, t_v, idx_v, buf_v, acc_v, sem):
    wid = lax.axis_index("s") * _NC + lax.axis_index("c")
    base = wid * _R
    pltpu.sync_copy(t_hbm.at[pl.ds(base, _R)], t_v)
    for k in range(_R // _L):
        tv = t_v[pl.ds(k * _L, _L)]
        rows = base + k * _L + lax.iota(jnp.int32, _L)
        strip = ((rows >> 3) * _TPR + (tv >> 7)) * 8 + (rows & 7)
        idx_v[pl.ds(k * _L, _L)] = strip
    pltpu.async_copy(xs_hbm.at[idx_v], buf_v, sem).wait()
    cf = jnp.float32(_CONF - _FILL)
    acc = jnp.zeros((_L,), jnp.float32)
    for k in range(_R // _L):
        tv = t_v[pl.ds(k * _L, _L)]
        acc = acc + jnp.where(tv != 0, jnp.float32(_ROW_K), 0.0)
        t127 = tv & 127
        for r2 in range(_L):
            r = k * _L + r2
            bc = jnp.full((_L,), t127[r2], jnp.int32)
            valid = tv[r2] != 0
            gval = jnp.zeros((_L,), jnp.float32)
            for c in range(8):
                seg = buf_v[r, pl.ds(c * _L, _L)]
                lane = lax.iota(jnp.int32, _L) + c * _L
                gval = gval + jnp.where(lane == bc, seg, 0.0)
            acc = acc - jnp.where(valid, cf * gval, 0.0)
    acc_v[...] = acc
    pltpu.sync_copy(acc_v, out_hbm.at[wid])


def _sc_row_terms(xs, target):
    mesh = plsc.VectorSubcoreMesh(core_axis_name="c", subcore_axis_name="s")
    fn = functools.partial(
        pl.kernel,
        mesh=mesh,
        out_type=jax.ShapeDtypeStruct((_NW, _L), jnp.float32),
        scratch_types=[
            pltpu.VMEM((_R,), jnp.int32),
            pltpu.VMEM((_R,), jnp.int32),
            pltpu.VMEM((_R, 128), jnp.float32),
            pltpu.VMEM((_L,), jnp.float32),
            pltpu.SemaphoreType.DMA,
        ],
    )(_sc_body)
    return fn(xs, target)


_BN = 128
_BV = 32000
_NBN = _N // _BN
_NBV = _SIZE // _BV


def _tc_body(t_ref, x_ref, out_ref, acc_ref):
    i = pl.program_id(0)
    j = pl.program_id(1)
    x = x_ref[...]

    @pl.when(j == 0)
    def _reset():
        lane = lax.broadcasted_iota(jnp.int32, (_BN, 128), 1)
        acc_ref[...] = jnp.where(lane == 0, -x[:, 0:128], 0.0)

    acc = acc_ref[...]
    for k in range(_BV // 128):
        acc += x[:, k * 128:(k + 1) * 128]
    acc_ref[...] = acc

    @pl.when((i == 0) & (j == 0))
    def _init():
        out_ref[0, 0] = 0.0

    @pl.when(j == _NBV - 1)
    def _finish():
        m = t_ref[0, 0, :][:, None] != 0
        masked = jnp.where(m, acc_ref[...], 0.0)
        out_ref[0, 0] += -_FILL * jnp.sum(masked)


def _tc_masked_sum(x, t3):
    out = pl.pallas_call(
        _tc_body,
        grid=(_NBN, _NBV),
        in_specs=[
            pl.BlockSpec((1, 1, _BN), lambda i, j: (i, 0, 0)),
            pl.BlockSpec((_BN, _BV), lambda i, j: (i, j)),
        ],
        out_specs=pl.BlockSpec((1, 1), lambda i, j: (0, 0),
                               memory_space=pltpu.SMEM),
        out_shape=jax.ShapeDtypeStruct((1, 1), jnp.float32),
        scratch_shapes=[pltpu.VMEM((_BN, 128), jnp.float32)],
        compiler_params=pltpu.CompilerParams(
            dimension_semantics=("arbitrary", "arbitrary"),
            vmem_limit_bytes=128 * 1024 * 1024,
        ),
    )(t3, x)
    return out[0, 0]


def kernel(x, target):
    n, size = x.shape
    assert (n, size) == (_N, _SIZE)
    t32 = target.astype(jnp.int32)
    xs = (x.reshape(_N // 8, 8, _TPR, 128)
           .swapaxes(1, 2)
           .reshape(_N // 8 * _TPR * 8, 128))
    parts = _sc_row_terms(xs, t32)
    t3 = t32.reshape(_NBN, 1, _BN)
    aprime = _tc_masked_sum(x, t3)
    return jnp.sum(parts) + aprime


# speedup vs baseline: 1.0269x; 1.0269x over previous
import functools
import math

import jax
import jax.numpy as jnp
from jax import lax
from jax.experimental import pallas as pl
from jax.experimental.pallas import tpu as pltpu
from jax.experimental.pallas import tpu_sc as plsc

_N = 2048
_SIZE = 32000
_SMOOTHING = 0.1
_CONF = 1.0 - _SMOOTHING
_FILL = _SMOOTHING / (_SIZE - 2)
_ROW_K = _CONF * math.log(_CONF) + (_SIZE - 2) * _FILL * math.log(_FILL)

_NC = 2
_NS = 16
_NW = _NC * _NS
_R = _N // _NW
_L = 16
_TPR = _SIZE // 128


def _sc_body(xs_hbm, t_hbm, out_hbm, t_v, idx_v, buf_v, acc_v, sem):
    wid = lax.axis_index("s") * _NC + lax.axis_index("c")
    base = wid * _R
    pltpu.sync_copy(t_hbm.at[pl.ds(base, _R)], t_v)
    for k in range(_R // _L):
        tv = t_v[pl.ds(k * _L, _L)]
        rows = base + k * _L + lax.iota(jnp.int32, _L)
        strip = ((rows >> 3) * _TPR + (tv >> 7)) * 8 + (rows & 7)
        idx_v[pl.ds(k * _L, _L)] = strip
    pltpu.async_copy(xs_hbm.at[idx_v], buf_v, sem).wait()
    cf = jnp.float32(_CONF - _FILL)
    acc = jnp.zeros((_L,), jnp.float32)
    for k in range(_R // _L):
        tv = t_v[pl.ds(k * _L, _L)]
        acc = acc + jnp.where(tv != 0, jnp.float32(_ROW_K), 0.0)
        t127 = tv & 127
        for r2 in range(_L):
            r = k * _L + r2
            bc = jnp.full((_L,), t127[r2], jnp.int32)
            valid = tv[r2] != 0
            gval = jnp.zeros((_L,), jnp.float32)
            for c in range(8):
                seg = buf_v[r, pl.ds(c * _L, _L)]
                lane = lax.iota(jnp.int32, _L) + c * _L
                gval = gval + jnp.where(lane == bc, seg, 0.0)
            acc = acc - jnp.where(valid, cf * gval, 0.0)
    acc_v[...] = acc
    pltpu.sync_copy(acc_v, out_hbm.at[wid])


def _sc_row_terms(xs, target):
    mesh = plsc.VectorSubcoreMesh(core_axis_name="c", subcore_axis_name="s")
    fn = functools.partial(
        pl.kernel,
        mesh=mesh,
        out_type=jax.ShapeDtypeStruct((_NW, _L), jnp.float32),
        scratch_types=[
            pltpu.VMEM((_R,), jnp.int32),
            pltpu.VMEM((_R,), jnp.int32),
            pltpu.VMEM((_R, 128), jnp.float32),
            pltpu.VMEM((_L,), jnp.float32),
            pltpu.SemaphoreType.DMA,
        ],
    )(_sc_body)
    return fn(xs, target)


_BN = 128
_BV = 32000
_NBN = _N // _BN
_NBV = _SIZE // _BV


def _tc_body(t_ref, x_ref, out_ref, acc_ref):
    i = pl.program_id(0)
    j = pl.program_id(1)
    x = x_ref[...]

    @pl.when(j == 0)
    def _reset():
        lane = lax.broadcasted_iota(jnp.int32, (_BN, 128), 1)
        acc_ref[...] = jnp.where(lane == 0, -x[:, 0:128], 0.0)

    acc = acc_ref[...]
    for k in range(_BV // 128):
        acc += x[:, k * 128:(k + 1) * 128]
    acc_ref[...] = acc

    @pl.when((i == 0) & (j == 0))
    def _init():
        out_ref[0, 0] = 0.0

    @pl.when(j == _NBV - 1)
    def _finish():
        m = t_ref[0, 0, :][:, None] != 0
        masked = jnp.where(m, acc_ref[...], 0.0)
        out_ref[0, 0] += -_FILL * jnp.sum(masked)


def _tc_masked_sum(x, t3):
    out = pl.pallas_call(
        _tc_body,
        grid=(_NBN, _NBV),
        in_specs=[
            pl.BlockSpec((1, 1, _BN), lambda i, j: (i, 0, 0)),
            pl.BlockSpec((_BN, _BV), lambda i, j: (i, j)),
        ],
        out_specs=pl.BlockSpec((1, 1), lambda i, j: (0, 0),
                               memory_space=pltpu.SMEM),
        out_shape=jax.ShapeDtypeStruct((1, 1), jnp.float32),
        scratch_shapes=[pltpu.VMEM((_BN, 128), jnp.float32)],
        compiler_params=pltpu.CompilerParams(
            dimension_semantics=("arbitrary", "arbitrary"),
        ),
    )(t3, x)
    return out[0, 0]


def kernel(x, target):
    n, size = x.shape
    assert (n, size) == (_N, _SIZE)
    t32 = target.astype(jnp.int32)
    xs = (x.reshape(_N // 8, 8, _TPR, 128)
           .swapaxes(1, 2)
           .reshape(_N // 8 * _TPR * 8, 128))
    parts = _sc_row_terms(xs, t32)
    t3 = t32.reshape(_NBN, 1, _BN)
    aprime = _tc_masked_sum(x, t3)
    return jnp.sum(parts) + aprime
